# baseline (speedup 1.0000x reference)
import functools

import jax
import jax.numpy as jnp
from jax.experimental import pallas as pl

N_NODES = 1024
N_BT = 384
D_IN = 2
D_OUT = 64
N_MAT = 5
TB = 12
ROWS = TB * D_IN


def _body(s_ref, x_ref, w_ref, b_ref, o_ref):
    xb = x_ref[...]
    s0 = s_ref[0]
    s1 = s_ref[1]
    dn = (((1,), (1,)), ((), ()))
    z1 = jax.lax.dot_general(xb, s0, dn)
    z2 = jax.lax.dot_general(z1, s0, dn)
    z3 = jax.lax.dot_general(xb, s1, dn)
    z4 = jax.lax.dot_general(z3, s1, dn)
    g = jnp.concatenate(
        [z.reshape(TB, D_IN, N_NODES) for z in (xb, z1, z2, z3, z4)], axis=1
    ).reshape(TB * N_MAT * D_IN, N_NODES)
    g = jnp.concatenate(
        [g, jnp.zeros((128 - TB * N_MAT * D_IN, N_NODES), jnp.float32)], axis=0
    )
    gt = g.T
    out12 = (
        jax.lax.dot_general(gt, w_ref[...], (((1,), (0,)), ((), ())))
        + b_ref[...]
    )
    for t in range(TB):
        o_ref[t] = out12[:, t * D_OUT : (t + 1) * D_OUT]


@jax.jit
def kernel(supports, x, weight, biases):
    w12 = jnp.kron(jnp.eye(TB, dtype=weight.dtype), weight)
    w12 = jnp.pad(w12, ((0, 128 - TB * N_MAT * D_IN), (0, 0)))
    b12 = jnp.tile(biases, (TB,)).reshape(1, TB * D_OUT)
    grid = N_BT // TB
    out = pl.pallas_call(
        _body,
        grid=(grid,),
        in_specs=[
            pl.BlockSpec((2, N_NODES, N_NODES), lambda i: (0, 0, 0)),
            pl.BlockSpec((ROWS, N_NODES), lambda i: (i, 0)),
            pl.BlockSpec((128, TB * D_OUT), lambda i: (0, 0)),
            pl.BlockSpec((1, TB * D_OUT), lambda i: (0, 0)),
        ],
        out_specs=pl.BlockSpec((TB, N_NODES, D_OUT), lambda i: (i, 0, 0)),
        out_shape=jax.ShapeDtypeStruct((N_BT, N_NODES, D_OUT), jnp.float32),
    )(supports, x, w12, b12)
    return out


# speedup vs baseline: 1.1817x; 1.1817x over previous
import jax
import jax.numpy as jnp
from jax.experimental import pallas as pl
from jax.experimental.pallas import tpu as pltpu

N_NODES = 1024
N_BT = 384
D_IN = 2
D_OUT = 64
N_MAT = 5
TB = 12
ROWS = TB * D_IN
GROWS = TB * N_MAT * D_IN


def _body(s_ref, x_ref, w_ref, o_ref, z1_ref, z2_ref, z3_ref, z4_ref):
    i = pl.program_id(0)

    @pl.when(i == 0)
    def _diffuse():
        xb = x_ref[...]
        s0 = s_ref[0]
        s1 = s_ref[1]
        dn = (((1,), (1,)), ((), ()))
        z1 = jax.lax.dot_general(xb, s0, dn)
        z1_ref[...] = z1
        z2_ref[...] = jax.lax.dot_general(z1, s0, dn)
        z3 = jax.lax.dot_general(xb, s1, dn)
        z3_ref[...] = z3
        z4_ref[...] = jax.lax.dot_general(z3, s1, dn)

    sl = pl.ds(i * ROWS, ROWS)
    pieces = (x_ref[sl, :], z1_ref[sl, :], z2_ref[sl, :],
              z3_ref[sl, :], z4_ref[sl, :])
    g = jnp.concatenate(
        [z.reshape(TB, D_IN, N_NODES) for z in pieces], axis=1
    ).reshape(GROWS, N_NODES)
    pad = jnp.concatenate(
        [jnp.ones((1, N_NODES), jnp.float32),
         jnp.zeros((128 - GROWS - 1, N_NODES), jnp.float32)], axis=0)
    g = jnp.concatenate([g, pad], axis=0)
    gt = g.T
    out12 = jax.lax.dot_general(gt, w_ref[...], (((1,), (0,)), ((), ())))
    for t in range(TB):
        o_ref[t] = out12[:, t * D_OUT : (t + 1) * D_OUT]


@jax.jit
def kernel(supports, x, weight, biases):
    w12 = jnp.kron(jnp.eye(TB, dtype=weight.dtype), weight)
    w12 = jnp.concatenate(
        [w12, jnp.tile(biases, (1, TB)),
         jnp.zeros((128 - GROWS - 1, TB * D_OUT), w12.dtype)], axis=0)
    grid = N_BT // TB
    out = pl.pallas_call(
        _body,
        grid=(grid,),
        in_specs=[
            pl.BlockSpec((2, N_NODES, N_NODES), lambda i: (0, 0, 0)),
            pl.BlockSpec((N_BT * D_IN, N_NODES), lambda i: (0, 0)),
            pl.BlockSpec((128, TB * D_OUT), lambda i: (0, 0)),
        ],
        out_specs=pl.BlockSpec((TB, N_NODES, D_OUT), lambda i: (i, 0, 0)),
        out_shape=jax.ShapeDtypeStruct((N_BT, N_NODES, D_OUT), jnp.float32),
        scratch_shapes=[pltpu.VMEM((N_BT * D_IN, N_NODES), jnp.float32)] * 4,
    )(supports, x, w12)
    return out
